# Initial kernel scaffold; baseline (speedup 1.0000x reference)
import jax
import jax.numpy as jnp
from jax.experimental import pallas as pl


def kernel(cls_time, y, fine_to_super_lut):
    raise NotImplementedError("write your pallas kernel here")


# speedup vs baseline: 10.0574x; 10.0574x over previous
import functools

import jax
import jax.numpy as jnp
from jax import lax
from jax.experimental import pallas as pl
from jax.experimental.pallas import tpu as pltpu
from jax.experimental.pallas import tpu_sc as plsc

BATCH = 16384
NUM_FINE = 100
FINE_PAD = 112
HIST_W = 2 * FINE_PAD
NUM_SUPER_PAD = 32
NW = 32
CHUNK = BATCH // NW
L = 16
MARGIN = 0.3


def _acosh16(x):
    one = jnp.float32(1.0)
    x = jnp.maximum(x, jnp.float32(1.001))
    u = x * x - one
    ui = lax.bitcast_convert_type(u, jnp.int32)
    r = lax.bitcast_convert_type(jnp.int32(0x5F3759DF) - (ui >> 1), jnp.float32)
    half_u = jnp.float32(0.5) * u
    for _ in range(3):
        r = r * (jnp.float32(1.5) - half_u * r * r)
    t = x + u * r
    ti = lax.bitcast_convert_type(t, jnp.int32)
    e = ((ti >> 23) & jnp.int32(255)) - jnp.int32(127)
    m = lax.bitcast_convert_type(
        (ti & jnp.int32(0x007FFFFF)) | jnp.int32(0x3F800000), jnp.float32)
    big = m > jnp.float32(1.4142135)
    m = jnp.where(big, m * jnp.float32(0.5), m)
    e = jnp.where(big, e + jnp.int32(1), e)
    q = (m - one) / (m + one)
    z = q * q
    p = jnp.float32(2.0) * q * (
        one + z * (jnp.float32(1.0 / 3.0) + z * (jnp.float32(0.2) + z * (
            jnp.float32(1.0 / 7.0) + z * jnp.float32(1.0 / 9.0)))))
    return e.astype(jnp.float32) * jnp.float32(0.6931471805599453) + p


def _sc_partials(x, y):
    mesh = plsc.VectorSubcoreMesh(core_axis_name="c", subcore_axis_name="s")

    @functools.partial(
        pl.kernel,
        out_type=jax.ShapeDtypeStruct((NW, HIST_W), jnp.float32),
        mesh=mesh,
        scratch_types=[
            pltpu.VMEM((CHUNK,), jnp.float32),
            pltpu.VMEM((CHUNK,), jnp.int32),
            pltpu.VMEM((L * HIST_W,), jnp.float32),
            pltpu.VMEM((HIST_W,), jnp.float32),
        ],
        compiler_params=pltpu.CompilerParams(needs_layout_passes=False),
    )
    def body(x_hbm, y_hbm, out_hbm, x_v, y_v, hist_v, row_v):
        cid = lax.axis_index("c")
        sid = lax.axis_index("s")
        wid = sid * 2 + cid
        base = wid * CHUNK
        pltpu.sync_copy(x_hbm.at[pl.ds(base, CHUNK)], x_v)
        pltpu.sync_copy(y_hbm.at[pl.ds(base, CHUNK)], y_v)

        zeros = jnp.zeros((L,), jnp.float32)
        for k in range(L * HIST_W // L):
            hist_v[pl.ds(k * L, L)] = zeros

        ones = jnp.ones((L,), jnp.float32)
        lane_base = lax.broadcasted_iota(jnp.int32, (L,), 0) * jnp.int32(HIST_W)
        for i in range(CHUNK // L):
            xv = x_v[pl.ds(i * L, L)]
            lbl = y_v[pl.ds(i * L, L)]
            d = _acosh16(xv)
            fidx = lane_base + lbl
            plsc.addupdate_scatter(hist_v, [fidx], d)
            plsc.addupdate_scatter(hist_v, [fidx + jnp.int32(FINE_PAD)], ones)

        for j in range(HIST_W // L):
            acc = hist_v[pl.ds(j * L, L)]
            for r in range(1, L):
                acc = acc + hist_v[pl.ds(r * HIST_W + j * L, L)]
            row_v[pl.ds(j * L, L)] = acc
        pltpu.sync_copy(row_v, out_hbm.at[wid])

    return body(x, y)


def _tc_body(p_ref, lut_ref, o_ref):
    tot = jnp.sum(p_ref[...], axis=0, keepdims=True)
    fine_sum = tot[:, :FINE_PAD]
    fine_count = tot[:, FINE_PAD:]
    fine_mean = fine_sum / jnp.maximum(fine_count, 1.0)
    mask_fine = (fine_count > 0).astype(jnp.float32)
    stacked = jnp.concatenate(
        [fine_sum, fine_count, fine_mean * mask_fine, mask_fine], axis=0)
    onehot = (lut_ref[...] == lax.broadcasted_iota(
        jnp.int32, (FINE_PAD, NUM_SUPER_PAD), 1)).astype(jnp.float32)
    seg = jax.lax.dot_general(
        stacked, onehot, (((1,), (0,)), ((), ())),
        preferred_element_type=jnp.float32)
    super_sum = seg[0:1]
    super_count = seg[1:2]
    fms_sum = seg[2:3]
    fcs = seg[3:4]
    super_mean = super_sum / jnp.maximum(super_count, 1.0)
    fine_mean_per_super = fms_sum / jnp.maximum(fcs, 1.0)
    mask = ((super_count > 0) & (fcs > 0)).astype(jnp.float32)
    hinge = jnp.maximum(super_mean - fine_mean_per_super + MARGIN, 0.0) ** 2
    msum = jnp.sum(mask)
    loss = jnp.where(msum > 0,
                     jnp.sum(hinge * mask) / jnp.maximum(msum, 1.0), 0.0)
    o_ref[...] = jnp.reshape(loss, (1, 1))


def kernel(cls_time, y, fine_to_super_lut):
    x = cls_time.reshape(-1)
    lut_pad = jnp.concatenate(
        [fine_to_super_lut,
         jnp.full((FINE_PAD - NUM_FINE,), NUM_SUPER_PAD - 1, jnp.int32)])
    partials = _sc_partials(x, y)
    loss = pl.pallas_call(
        _tc_body,
        out_shape=jax.ShapeDtypeStruct((1, 1), jnp.float32),
    )(partials, lut_pad.reshape(FINE_PAD, 1))
    return loss[0, 0]


# speedup vs baseline: 10.5982x; 1.0538x over previous
import functools

import jax
import jax.numpy as jnp
from jax import lax
from jax.experimental import pallas as pl
from jax.experimental.pallas import tpu as pltpu
from jax.experimental.pallas import tpu_sc as plsc

BATCH = 16384
NUM_FINE = 100
FINE_PAD = 112
HIST_W = 2 * FINE_PAD
NUM_SUPER_PAD = 32
NW = 32
CHUNK = BATCH // NW
L = 16
MARGIN = 0.3


def _acosh16(x):
    one = jnp.float32(1.0)
    x = jnp.maximum(x, jnp.float32(1.001))
    u = x * x - one
    ui = lax.bitcast_convert_type(u, jnp.int32)
    r = lax.bitcast_convert_type(jnp.int32(0x5F3759DF) - (ui >> 1), jnp.float32)
    half_u = jnp.float32(0.5) * u
    for _ in range(3):
        r = r * (jnp.float32(1.5) - half_u * r * r)
    t = x + u * r
    ti = lax.bitcast_convert_type(t, jnp.int32)
    e = ((ti >> 23) & jnp.int32(255)) - jnp.int32(127)
    m = lax.bitcast_convert_type(
        (ti & jnp.int32(0x007FFFFF)) | jnp.int32(0x3F800000), jnp.float32)
    big = m > jnp.float32(1.4142135)
    m = jnp.where(big, m * jnp.float32(0.5), m)
    e = jnp.where(big, e + jnp.int32(1), e)
    q = (m - one) / (m + one)
    z = q * q
    p = jnp.float32(2.0) * q * (
        one + z * (jnp.float32(1.0 / 3.0) + z * (jnp.float32(0.2) + z * (
            jnp.float32(1.0 / 7.0) + z * jnp.float32(1.0 / 9.0)))))
    return e.astype(jnp.float32) * jnp.float32(0.6931471805599453) + p


def _sc_partials(x, y):
    mesh = plsc.VectorSubcoreMesh(core_axis_name="c", subcore_axis_name="s")

    @functools.partial(
        pl.kernel,
        out_type=jax.ShapeDtypeStruct((NW, HIST_W), jnp.float32),
        mesh=mesh,
        scratch_types=[
            pltpu.VMEM((CHUNK,), jnp.float32),
            pltpu.VMEM((CHUNK,), jnp.int32),
            pltpu.VMEM((HIST_W,), jnp.float32),
        ],
        compiler_params=pltpu.CompilerParams(needs_layout_passes=False),
    )
    def body(x_hbm, y_hbm, out_hbm, x_v, y_v, hist_v):
        cid = lax.axis_index("c")
        sid = lax.axis_index("s")
        wid = sid * 2 + cid
        base = wid * CHUNK
        pltpu.sync_copy(x_hbm.at[pl.ds(base, CHUNK)], x_v)
        pltpu.sync_copy(y_hbm.at[pl.ds(base, CHUNK)], y_v)

        zeros = jnp.zeros((L,), jnp.float32)
        for k in range(HIST_W // L):
            hist_v[pl.ds(k * L, L)] = zeros

        ones = jnp.ones((L,), jnp.float32)
        for i in range(CHUNK // L):
            xv = x_v[pl.ds(i * L, L)]
            lbl = y_v[pl.ds(i * L, L)]
            d = _acosh16(xv)
            plsc.addupdate_scatter(hist_v, [lbl], d)
            plsc.addupdate_scatter(hist_v, [lbl + jnp.int32(FINE_PAD)], ones)

        pltpu.sync_copy(hist_v, out_hbm.at[wid])

    return body(x, y)


def _tc_body(p_ref, lut_ref, o_ref):
    tot = jnp.sum(p_ref[...], axis=0, keepdims=True)
    fine_sum = tot[:, :FINE_PAD]
    fine_count = tot[:, FINE_PAD:]
    fine_mean = fine_sum / jnp.maximum(fine_count, 1.0)
    mask_fine = (fine_count > 0).astype(jnp.float32)
    stacked = jnp.concatenate(
        [fine_sum, fine_count, fine_mean * mask_fine, mask_fine], axis=0)
    onehot = (lut_ref[...] == lax.broadcasted_iota(
        jnp.int32, (FINE_PAD, NUM_SUPER_PAD), 1)).astype(jnp.float32)
    seg = jax.lax.dot_general(
        stacked, onehot, (((1,), (0,)), ((), ())),
        preferred_element_type=jnp.float32)
    super_sum = seg[0:1]
    super_count = seg[1:2]
    fms_sum = seg[2:3]
    fcs = seg[3:4]
    super_mean = super_sum / jnp.maximum(super_count, 1.0)
    fine_mean_per_super = fms_sum / jnp.maximum(fcs, 1.0)
    mask = ((super_count > 0) & (fcs > 0)).astype(jnp.float32)
    hinge = jnp.maximum(super_mean - fine_mean_per_super + MARGIN, 0.0) ** 2
    msum = jnp.sum(mask)
    loss = jnp.where(msum > 0,
                     jnp.sum(hinge * mask) / jnp.maximum(msum, 1.0), 0.0)
    o_ref[...] = jnp.reshape(loss, (1, 1))


def kernel(cls_time, y, fine_to_super_lut):
    x = cls_time.reshape(-1)
    lut_pad = jnp.concatenate(
        [fine_to_super_lut,
         jnp.full((FINE_PAD - NUM_FINE,), NUM_SUPER_PAD - 1, jnp.int32)])
    partials = _sc_partials(x, y)
    loss = pl.pallas_call(
        _tc_body,
        out_shape=jax.ShapeDtypeStruct((1, 1), jnp.float32),
    )(partials, lut_pad.reshape(FINE_PAD, 1))
    return loss[0, 0]
